# Initial kernel scaffold; baseline (speedup 1.0000x reference)
import jax
import jax.numpy as jnp
from jax.experimental import pallas as pl


def kernel(x, edge_index, size_factors, W1, b1, ln_g, ln_b, W2, b2):
    raise NotImplementedError("write your pallas kernel here")


# speedup vs baseline: 19.6710x; 19.6710x over previous
import functools

import jax
import jax.numpy as jnp
from jax import lax
from jax.experimental import pallas as pl
from jax.experimental.pallas import tpu as pltpu
from jax.experimental.pallas import tpu_sc as plsc

_N = 10000
_D = 128
_H = 2
_E = 320000

_N_PAD = 10240
_NW = 32
_EPT = 10240
_E_PAD = _NW * _EPT
_CH1 = 80
_CH2 = 160
_TPW = _N_PAD // 16

_mesh = plsc.VectorSubcoreMesh(core_axis_name="c", subcore_axis_name="s")


@functools.partial(
    pl.kernel,
    out_type=jax.ShapeDtypeStruct((2 * _N_PAD,), jnp.float32),
    mesh=_mesh,
    scratch_types=[
        pltpu.VMEM((_CH1, 128), jnp.int32),
        pltpu.VMEM((128,), jnp.float32),
        pltpu.VMEM_SHARED((_N_PAD,), jnp.float32),
    ],
)
def _sc_degree(dstr_hbm, zeros_hbm, out_hbm, dst_v, ones_v, cnt_sp):
    cid = lax.axis_index("c")
    sid = lax.axis_index("s")
    wid = sid * 2 + cid
    r0 = sid * _TPW
    pltpu.sync_copy(zeros_hbm.at[pl.ds(r0, _TPW)], cnt_sp.at[pl.ds(r0, _TPW)])
    pltpu.sync_copy(dstr_hbm.at[wid], dst_v)
    for k in range(8):
        ones_v[pl.ds(k * 16, 16)] = jnp.ones((16,), jnp.float32)
    plsc.subcore_barrier()

    def body(j, carry):
        pltpu.sync_copy(ones_v, cnt_sp.at[dst_v.at[j]], add=True)
        return carry

    lax.fori_loop(0, _CH1, body, 0)
    plsc.subcore_barrier()
    pltpu.sync_copy(cnt_sp.at[pl.ds(r0, _TPW)],
                    out_hbm.at[pl.ds(cid * _N_PAD + r0, _TPW)])


@functools.partial(
    pl.kernel,
    out_type=jax.ShapeDtypeStruct((4 * _N_PAD,), jnp.float32),
    mesh=_mesh,
    scratch_types=[
        pltpu.VMEM((_CH2, 128), jnp.int32),
        pltpu.VMEM((_CH2, 128), jnp.int32),
        pltpu.VMEM((128,), jnp.float32),
        pltpu.VMEM_SHARED((2 * _N_PAD,), jnp.float32),
        pltpu.SemaphoreType.DMA,
    ],
)
def _sc_edge_agg(y_hbm, src2_hbm, dst2_hbm, zeros_hbm, out_hbm,
                 src_v, dst_v, upd_v, z_sp, gsem):
    cid = lax.axis_index("c")
    sid = lax.axis_index("s")
    wid = sid * 2 + cid
    f0 = sid * (2 * _TPW)
    pltpu.sync_copy(zeros_hbm.at[pl.ds(f0, 2 * _TPW)], z_sp.at[pl.ds(f0, 2 * _TPW)])
    pltpu.sync_copy(src2_hbm.at[wid], src_v)
    pltpu.sync_copy(dst2_hbm.at[wid], dst_v)
    plsc.subcore_barrier()

    def body(j, carry):
        pltpu.async_copy(y_hbm.at[src_v.at[j]], upd_v, gsem).wait()
        pltpu.sync_copy(upd_v, z_sp.at[dst_v.at[j]], add=True)
        return carry

    lax.fori_loop(0, _CH2, body, 0)
    plsc.subcore_barrier()
    pltpu.sync_copy(z_sp.at[pl.ds(f0, 2 * _TPW)],
                    out_hbm.at[pl.ds(cid * 2 * _N_PAD + f0, 2 * _TPW)])


_BLK = 1000


def _tc1_body(x_ref, prm_ref, d0_ref, d1_ref, y1_ref, dinv_ref):
    prm = prm_ref[...]
    xb = x_ref[...]
    dinv = lax.rsqrt(d0_ref[...] + d1_ref[...] + 1.0)
    h0 = jnp.sum(xb * prm[0:1, :], axis=1, keepdims=True)
    h1 = jnp.sum(xb * prm[1:2, :], axis=1, keepdims=True)
    y1_ref[...] = jnp.concatenate([h0, h1], axis=1) * dinv
    dinv_ref[...] = dinv


_tc1 = pl.pallas_call(
    _tc1_body,
    grid=(_N // _BLK,),
    in_specs=[
        pl.BlockSpec((_BLK, _D), lambda i: (i, 0)),
        pl.BlockSpec((8, _D), lambda i: (0, 0)),
        pl.BlockSpec((_BLK, 1), lambda i: (i, 0)),
        pl.BlockSpec((_BLK, 1), lambda i: (i, 0)),
    ],
    out_specs=[
        pl.BlockSpec((_BLK, _H), lambda i: (i, 0)),
        pl.BlockSpec((_BLK, 1), lambda i: (i, 0)),
    ],
    out_shape=[
        jax.ShapeDtypeStruct((_N, _H), jnp.float32),
        jax.ShapeDtypeStruct((_N, 1), jnp.float32),
    ],
)


def _tc2_body(z1a_ref, z1b_ref, y1_ref, dinv_ref, prm_ref, y2_ref):
    prm = prm_ref[...]
    g = prm[5:6, 0:_H]
    bln = prm[6:7, 0:_H]
    b1r = prm[7:8, 0:_H]
    dinv = dinv_ref[...]
    out1 = dinv * (z1a_ref[...] + z1b_ref[...] + y1_ref[...]) + b1r
    mu = jnp.mean(out1, axis=1, keepdims=True)
    var = jnp.mean((out1 - mu) ** 2, axis=1, keepdims=True)
    a = (out1 - mu) * lax.rsqrt(var + 1e-5) * g + bln
    y2_ref[...] = jnp.maximum(a, 0.0) * dinv


_tc2 = pl.pallas_call(
    _tc2_body,
    grid=(_N // _BLK,),
    in_specs=[
        pl.BlockSpec((_BLK, _H), lambda i: (i, 0)),
        pl.BlockSpec((_BLK, _H), lambda i: (i, 0)),
        pl.BlockSpec((_BLK, _H), lambda i: (i, 0)),
        pl.BlockSpec((_BLK, 1), lambda i: (i, 0)),
        pl.BlockSpec((8, _D), lambda i: (0, 0)),
    ],
    out_specs=pl.BlockSpec((_BLK, _H), lambda i: (i, 0)),
    out_shape=jax.ShapeDtypeStruct((_N, _H), jnp.float32),
)


def _tc3_body(z2a_ref, z2b_ref, y2_ref, dinv_ref, prm_ref, sf_ref, o_ref):
    prm = prm_ref[...]
    p = dinv_ref[...] * (z2a_ref[...] + z2b_ref[...] + y2_ref[...])
    h = p[:, 0:1] * prm[2:3, :] + p[:, 1:2] * prm[3:4, :] + prm[4:5, :]
    o_ref[...] = jnp.maximum(h, 0.0) * sf_ref[...]


_tc3 = pl.pallas_call(
    _tc3_body,
    grid=(_N // _BLK,),
    in_specs=[
        pl.BlockSpec((_BLK, _H), lambda i: (i, 0)),
        pl.BlockSpec((_BLK, _H), lambda i: (i, 0)),
        pl.BlockSpec((_BLK, _H), lambda i: (i, 0)),
        pl.BlockSpec((_BLK, 1), lambda i: (i, 0)),
        pl.BlockSpec((8, _D), lambda i: (0, 0)),
        pl.BlockSpec((_BLK, 1), lambda i: (i, 0)),
    ],
    out_specs=pl.BlockSpec((_BLK, _D), lambda i: (i, 0)),
    out_shape=jax.ShapeDtypeStruct((_N, _D), jnp.float32),
)


def kernel(x, edge_index, size_factors, W1, b1, ln_g, ln_b, W2, b2):
    src, dst = edge_index[0], edge_index[1]

    padn = _E_PAD - _E
    pad_idx = _N + (jnp.arange(padn, dtype=jnp.int32) % 128)
    src_p = jnp.concatenate([src, pad_idx])
    dst_p = jnp.concatenate([dst, pad_idx])
    dstr = dst_p.reshape(_NW, _CH1, 128)
    src2 = (src_p[:, None] * 2 + jnp.arange(2, dtype=jnp.int32)
            ).reshape(_NW, _CH2, 128)
    dst2 = (dst_p[:, None] * 2 + jnp.arange(2, dtype=jnp.int32)
            ).reshape(_NW, _CH2, 128)

    zeros = jnp.zeros((2 * _N_PAD,), jnp.float32)

    def pad128(v):
        return jnp.zeros((_D,), jnp.float32).at[: v.shape[0]].set(v)

    prm = jnp.stack([
        W1[:, 0], W1[:, 1],
        W2[0], W2[1],
        b2,
        pad128(ln_g), pad128(ln_b), pad128(b1),
    ])

    hist = _sc_degree(dstr, zeros[:_N_PAD])
    deg0 = hist[:_N][:, None]
    deg1 = hist[_N_PAD:_N_PAD + _N][:, None]

    y1, dinv = _tc1(x, prm, deg0, deg1)

    pad_rows = jnp.zeros((_N_PAD - _N, _H), jnp.float32)

    def agg(y):
        y_flat = jnp.concatenate([y, pad_rows]).reshape(-1)
        z = _sc_edge_agg(y_flat, src2, dst2, zeros)
        za = z[: 2 * _N_PAD].reshape(_N_PAD, _H)[:_N]
        zb = z[2 * _N_PAD:].reshape(_N_PAD, _H)[:_N]
        return za, zb

    z1a, z1b = agg(y1)
    y2 = _tc2(z1a, z1b, y1, dinv, prm)

    z2a, z2b = agg(y2)
    return _tc3(z2a, z2b, y2, dinv, prm, size_factors)


# speedup vs baseline: 32.4096x; 1.6476x over previous
import functools

import jax
import jax.numpy as jnp
from jax import lax
from jax.experimental import pallas as pl
from jax.experimental.pallas import tpu as pltpu
from jax.experimental.pallas import tpu_sc as plsc

_N = 10000
_D = 128
_H = 2
_E = 320000

_N_PAD = 10240
_NW = 32
_EPT = 10240
_E_PAD = _NW * _EPT
_CH1 = 80
_CH2 = 160
_TPW = _N_PAD // 16

_mesh = plsc.VectorSubcoreMesh(core_axis_name="c", subcore_axis_name="s")


@functools.partial(
    pl.kernel,
    out_type=jax.ShapeDtypeStruct((2 * _N_PAD,), jnp.float32),
    mesh=_mesh,
    scratch_types=[
        pltpu.VMEM((_CH1, 128), jnp.int32),
        pltpu.VMEM((128,), jnp.float32),
        pltpu.VMEM_SHARED((_N_PAD,), jnp.float32),
    ],
)
def _sc_degree(dstr_hbm, zeros_hbm, out_hbm, dst_v, ones_v, cnt_sp):
    cid = lax.axis_index("c")
    sid = lax.axis_index("s")
    wid = sid * 2 + cid
    r0 = sid * _TPW
    pltpu.sync_copy(zeros_hbm.at[pl.ds(r0, _TPW)], cnt_sp.at[pl.ds(r0, _TPW)])
    pltpu.sync_copy(dstr_hbm.at[wid], dst_v)
    for k in range(8):
        ones_v[pl.ds(k * 16, 16)] = jnp.ones((16,), jnp.float32)
    plsc.subcore_barrier()

    def body(j, carry):
        pltpu.sync_copy(ones_v, cnt_sp.at[dst_v.at[j]], add=True)
        return carry

    lax.fori_loop(0, _CH1, body, 0)
    plsc.subcore_barrier()
    pltpu.sync_copy(cnt_sp.at[pl.ds(r0, _TPW)],
                    out_hbm.at[pl.ds(cid * _N_PAD + r0, _TPW)])


@functools.partial(
    pl.kernel,
    out_type=jax.ShapeDtypeStruct((4 * _N_PAD,), jnp.float32),
    mesh=_mesh,
    scratch_types=[
        pltpu.VMEM((_CH2, 128), jnp.int32),
        pltpu.VMEM((_CH2, 128), jnp.int32),
        pltpu.VMEM((4, 128), jnp.float32),
        pltpu.VMEM_SHARED((2 * _N_PAD,), jnp.float32),
        pltpu.SemaphoreType.DMA,
        pltpu.SemaphoreType.DMA,
        pltpu.SemaphoreType.DMA,
        pltpu.SemaphoreType.DMA,
    ],
)
def _sc_edge_agg(y_hbm, src2_hbm, dst2_hbm, zeros_hbm, out_hbm,
                 src_v, dst_v, upd_v, z_sp, g0, g1, g2, g3):
    cid = lax.axis_index("c")
    sid = lax.axis_index("s")
    wid = sid * 2 + cid
    f0 = sid * (2 * _TPW)
    gsems = (g0, g1, g2, g3)
    pltpu.sync_copy(zeros_hbm.at[pl.ds(f0, 2 * _TPW)], z_sp.at[pl.ds(f0, 2 * _TPW)])
    pltpu.sync_copy(src2_hbm.at[wid], src_v)
    pltpu.sync_copy(dst2_hbm.at[wid], dst_v)
    plsc.subcore_barrier()

    def fire(c, b):
        pltpu.async_copy(y_hbm.at[src_v.at[c]], upd_v.at[b], gsems[b])

    def drain(b):
        pltpu.make_async_copy(y_hbm.at[src_v.at[0]], upd_v.at[b],
                              gsems[b]).wait()

    for b in range(4):
        fire(b, b)

    def body(i, carry):
        c0 = i * 4
        for b in range(4):
            drain(b)
            pltpu.sync_copy(upd_v.at[b], z_sp.at[dst_v.at[c0 + b]], add=True)
            fire(c0 + b + 4, b)
        return carry

    lax.fori_loop(0, _CH2 // 4 - 1, body, 0)
    c0 = _CH2 - 4
    for b in range(4):
        drain(b)
        pltpu.sync_copy(upd_v.at[b], z_sp.at[dst_v.at[c0 + b]], add=True)
    plsc.subcore_barrier()
    pltpu.sync_copy(z_sp.at[pl.ds(f0, 2 * _TPW)],
                    out_hbm.at[pl.ds(cid * 2 * _N_PAD + f0, 2 * _TPW)])


_BLK = 1000


def _tc1_body(x_ref, prm_ref, d0_ref, d1_ref, y1_ref, dinv_ref):
    prm = prm_ref[...]
    xb = x_ref[...]
    dinv = lax.rsqrt(d0_ref[...] + d1_ref[...] + 1.0)
    h0 = jnp.sum(xb * prm[0:1, :], axis=1, keepdims=True)
    h1 = jnp.sum(xb * prm[1:2, :], axis=1, keepdims=True)
    y1_ref[...] = jnp.concatenate([h0, h1], axis=1) * dinv
    dinv_ref[...] = dinv


_tc1 = pl.pallas_call(
    _tc1_body,
    grid=(_N // _BLK,),
    in_specs=[
        pl.BlockSpec((_BLK, _D), lambda i: (i, 0)),
        pl.BlockSpec((8, _D), lambda i: (0, 0)),
        pl.BlockSpec((_BLK, 1), lambda i: (i, 0)),
        pl.BlockSpec((_BLK, 1), lambda i: (i, 0)),
    ],
    out_specs=[
        pl.BlockSpec((_BLK, _H), lambda i: (i, 0)),
        pl.BlockSpec((_BLK, 1), lambda i: (i, 0)),
    ],
    out_shape=[
        jax.ShapeDtypeStruct((_N, _H), jnp.float32),
        jax.ShapeDtypeStruct((_N, 1), jnp.float32),
    ],
)


def _tc2_body(z1a_ref, z1b_ref, y1_ref, dinv_ref, prm_ref, y2_ref):
    prm = prm_ref[...]
    g = prm[5:6, 0:_H]
    bln = prm[6:7, 0:_H]
    b1r = prm[7:8, 0:_H]
    dinv = dinv_ref[...]
    out1 = dinv * (z1a_ref[...] + z1b_ref[...] + y1_ref[...]) + b1r
    mu = jnp.mean(out1, axis=1, keepdims=True)
    var = jnp.mean((out1 - mu) ** 2, axis=1, keepdims=True)
    a = (out1 - mu) * lax.rsqrt(var + 1e-5) * g + bln
    y2_ref[...] = jnp.maximum(a, 0.0) * dinv


_tc2 = pl.pallas_call(
    _tc2_body,
    grid=(_N // _BLK,),
    in_specs=[
        pl.BlockSpec((_BLK, _H), lambda i: (i, 0)),
        pl.BlockSpec((_BLK, _H), lambda i: (i, 0)),
        pl.BlockSpec((_BLK, _H), lambda i: (i, 0)),
        pl.BlockSpec((_BLK, 1), lambda i: (i, 0)),
        pl.BlockSpec((8, _D), lambda i: (0, 0)),
    ],
    out_specs=pl.BlockSpec((_BLK, _H), lambda i: (i, 0)),
    out_shape=jax.ShapeDtypeStruct((_N, _H), jnp.float32),
)


def _tc3_body(z2a_ref, z2b_ref, y2_ref, dinv_ref, prm_ref, sf_ref, o_ref):
    prm = prm_ref[...]
    p = dinv_ref[...] * (z2a_ref[...] + z2b_ref[...] + y2_ref[...])
    h = p[:, 0:1] * prm[2:3, :] + p[:, 1:2] * prm[3:4, :] + prm[4:5, :]
    o_ref[...] = jnp.maximum(h, 0.0) * sf_ref[...]


_tc3 = pl.pallas_call(
    _tc3_body,
    grid=(_N // _BLK,),
    in_specs=[
        pl.BlockSpec((_BLK, _H), lambda i: (i, 0)),
        pl.BlockSpec((_BLK, _H), lambda i: (i, 0)),
        pl.BlockSpec((_BLK, _H), lambda i: (i, 0)),
        pl.BlockSpec((_BLK, 1), lambda i: (i, 0)),
        pl.BlockSpec((8, _D), lambda i: (0, 0)),
        pl.BlockSpec((_BLK, 1), lambda i: (i, 0)),
    ],
    out_specs=pl.BlockSpec((_BLK, _D), lambda i: (i, 0)),
    out_shape=jax.ShapeDtypeStruct((_N, _D), jnp.float32),
)


def kernel(x, edge_index, size_factors, W1, b1, ln_g, ln_b, W2, b2):
    src, dst = edge_index[0], edge_index[1]

    padn = _E_PAD - _E
    pad_idx = _N + (jnp.arange(padn, dtype=jnp.int32) % 128)
    src_p = jnp.concatenate([src, pad_idx])
    dst_p = jnp.concatenate([dst, pad_idx])
    dstr = dst_p.reshape(_NW, _CH1, 128)
    src2 = (src_p[:, None] * 2 + jnp.arange(2, dtype=jnp.int32)
            ).reshape(_NW, _CH2, 128)
    dst2 = (dst_p[:, None] * 2 + jnp.arange(2, dtype=jnp.int32)
            ).reshape(_NW, _CH2, 128)

    zeros = jnp.zeros((2 * _N_PAD,), jnp.float32)

    def pad128(v):
        return jnp.zeros((_D,), jnp.float32).at[: v.shape[0]].set(v)

    prm = jnp.stack([
        W1[:, 0], W1[:, 1],
        W2[0], W2[1],
        b2,
        pad128(ln_g), pad128(ln_b), pad128(b1),
    ])

    hist = _sc_degree(dstr, zeros[:_N_PAD])
    deg0 = hist[:_N][:, None]
    deg1 = hist[_N_PAD:_N_PAD + _N][:, None]

    y1, dinv = _tc1(x, prm, deg0, deg1)

    pad_rows = jnp.zeros((_N_PAD - _N, _H), jnp.float32)

    def agg(y):
        y_flat = jnp.concatenate([y, pad_rows]).reshape(-1)
        z = _sc_edge_agg(y_flat, src2, dst2, zeros)
        za = z[: 2 * _N_PAD].reshape(_N_PAD, _H)[:_N]
        zb = z[2 * _N_PAD:].reshape(_N_PAD, _H)[:_N]
        return za, zb

    z1a, z1b = agg(y1)
    y2 = _tc2(z1a, z1b, y1, dinv, prm)

    z2a, z2b = agg(y2)
    return _tc3(z2a, z2b, y2, dinv, prm, size_factors)
